# Initial kernel scaffold; baseline (speedup 1.0000x reference)
import jax
import jax.numpy as jnp
from jax.experimental import pallas as pl


def kernel(x, edge_index, W, b, bn_g, bn_b, bn1_g, bn1_b, W1, b1, W2, b2, bn2_g, bn2_b):
    raise NotImplementedError("write your pallas kernel here")


# speedup vs baseline: 18.5961x; 18.5961x over previous
import functools

import jax
import jax.numpy as jnp
from jax import lax
from jax.experimental import pallas as pl
from jax.experimental.pallas import tpu as pltpu
from jax.experimental.pallas import tpu_sc as plsc

N = 10000
E = 320000
D = 128
DFF = 256
EPS = 1e-5

NC = 2
NS = 16
NW = NC * NS
EPW = E // NW
CH = 128
NFULL = EPW // CH
TAIL = EPW - NFULL * CH
RPT = 632
NROW = NS * RPT
DEG_CH = 640
NPAD = NS * DEG_CH

_mesh = plsc.VectorSubcoreMesh(core_axis_name="c", subcore_axis_name="s")


@functools.partial(
    pl.kernel,
    out_type=jax.ShapeDtypeStruct((NC * NPAD,), jnp.float32),
    mesh=_mesh,
    scratch_types=[
        pltpu.VMEM((CH,), jnp.int32),
        pltpu.VMEM((TAIL,), jnp.int32),
        pltpu.VMEM((CH,), jnp.float32),
        pltpu.VMEM_SHARED((NPAD,), jnp.float32),
    ],
)
def _deg_kernel(dst_hbm, zeros_hbm, out_hbm, idx_v, idxt_v, ones_v, acc):
    c = lax.axis_index("c")
    s = lax.axis_index("s")
    wid = s * NC + c

    def fill(i, carry):
        ones_v[pl.ds(i * 16, 16)] = jnp.ones((16,), jnp.float32)
        return carry

    lax.fori_loop(0, CH // 16, fill, 0)

    pltpu.sync_copy(zeros_hbm.at[pl.ds(s * DEG_CH, DEG_CH)],
                    acc.at[pl.ds(s * DEG_CH, DEG_CH)])
    plsc.subcore_barrier()
    base0 = wid * EPW

    def body(i, carry):
        base = base0 + i * CH
        pltpu.sync_copy(dst_hbm.at[pl.ds(base, CH)], idx_v)
        pltpu.sync_copy(ones_v, acc.at[idx_v], add=True)
        return carry

    lax.fori_loop(0, NFULL, body, 0)
    pltpu.sync_copy(dst_hbm.at[pl.ds(base0 + NFULL * CH, TAIL)], idxt_v)
    pltpu.sync_copy(ones_v.at[pl.ds(0, TAIL)], acc.at[idxt_v], add=True)
    plsc.subcore_barrier()
    pltpu.sync_copy(acc.at[pl.ds(s * DEG_CH, DEG_CH)],
                    out_hbm.at[pl.ds(c * NPAD + s * DEG_CH, DEG_CH)])


@functools.partial(
    pl.kernel,
    out_type=jax.ShapeDtypeStruct((NC, NROW, D), jnp.float32),
    mesh=_mesh,
    scratch_types=[
        pltpu.VMEM((CH,), jnp.int32),
        pltpu.VMEM((CH,), jnp.int32),
        pltpu.VMEM((TAIL,), jnp.int32),
        pltpu.VMEM((TAIL,), jnp.int32),
        pltpu.VMEM((CH, D), jnp.float32),
        pltpu.VMEM((TAIL, D), jnp.float32),
        pltpu.SemaphoreType.DMA,
        pltpu.VMEM_SHARED((NROW, D), jnp.float32),
    ],
)
def _agg_kernel(y_hbm, src_hbm, dst_hbm, zeros_hbm, out_hbm,
                sidx, didx, sidxt, didxt, rows, rowst, sem, acc):
    c = lax.axis_index("c")
    s = lax.axis_index("s")
    wid = s * NC + c

    pltpu.sync_copy(zeros_hbm.at[pl.ds(s * RPT, RPT)],
                    acc.at[pl.ds(s * RPT, RPT)])
    plsc.subcore_barrier()
    base0 = wid * EPW

    def body(i, carry):
        base = base0 + i * CH
        pltpu.sync_copy(src_hbm.at[pl.ds(base, CH)], sidx)
        pltpu.sync_copy(dst_hbm.at[pl.ds(base, CH)], didx)
        pltpu.async_copy(y_hbm.at[sidx], rows, sem).wait()
        pltpu.sync_copy(rows, acc.at[didx], add=True)
        return carry

    lax.fori_loop(0, NFULL, body, 0)
    tb = base0 + NFULL * CH
    pltpu.sync_copy(src_hbm.at[pl.ds(tb, TAIL)], sidxt)
    pltpu.sync_copy(dst_hbm.at[pl.ds(tb, TAIL)], didxt)
    pltpu.async_copy(y_hbm.at[sidxt], rowst, sem).wait()
    pltpu.sync_copy(rowst, acc.at[didxt], add=True)
    plsc.subcore_barrier()
    pltpu.sync_copy(acc.at[pl.ds(s * RPT, RPT)],
                    out_hbm.at[c, pl.ds(s * RPT, RPT)])


def _scale_body(deg0_ref, deg1_ref, x_ref, y_ref, dinv_ref):
    deg = deg0_ref[...] + deg1_ref[...] + 1.0
    dinv = lax.rsqrt(deg)
    dinv_ref[...] = dinv
    y_ref[...] = x_ref[...] * dinv


_scale_call = pl.pallas_call(
    _scale_body,
    out_shape=(
        jax.ShapeDtypeStruct((N, D), jnp.float32),
        jax.ShapeDtypeStruct((N, 1), jnp.float32),
    ),
)


def _bn(h, g, b):
    mu = jnp.mean(h, axis=0, keepdims=True)
    var = jnp.mean((h - mu) ** 2, axis=0, keepdims=True)
    return (h - mu) * lax.rsqrt(var + EPS) * g + b


def _dense_body(p0_ref, p1_ref, y_ref, dinv_ref, x_ref, W_ref, b_ref,
                bn_g_ref, bn_b_ref, bn1_g_ref, bn1_b_ref, W1_ref, b1_ref,
                W2_ref, b2_ref, bn2_g_ref, bn2_b_ref, out_ref):
    agg = (p0_ref[...] + p1_ref[...] + y_ref[...]) * dinv_ref[...]
    conv = jnp.dot(agg, W_ref[...], preferred_element_type=jnp.float32)
    conv = conv + b_ref[...]
    h = _bn(conv, bn_g_ref[...], bn_b_ref[...])
    h = jnp.maximum(h, 0.0) + x_ref[...]
    z = _bn(h, bn1_g_ref[...], bn1_b_ref[...])
    z = jnp.dot(z, W1_ref[...], preferred_element_type=jnp.float32) + b1_ref[...]
    z = jnp.maximum(z, 0.0)
    z = jnp.dot(z, W2_ref[...], preferred_element_type=jnp.float32) + b2_ref[...]
    h = h + z
    out_ref[...] = _bn(h, bn2_g_ref[...], bn2_b_ref[...])


_dense_call = pl.pallas_call(
    _dense_body,
    out_shape=jax.ShapeDtypeStruct((N, D), jnp.float32),
)


def kernel(x, edge_index, W, b, bn_g, bn_b, bn1_g, bn1_b,
           W1, b1, W2, b2, bn2_g, bn2_b):
    src = edge_index[0]
    dst = edge_index[1]
    zeros_vec = jnp.zeros((NPAD,), jnp.float32)
    zeros_mat = jnp.zeros((NROW, D), jnp.float32)

    degp = _deg_kernel(dst, zeros_vec)
    deg0 = degp[:N].reshape(N, 1)
    deg1 = degp[NPAD:NPAD + N].reshape(N, 1)
    y, dinv = _scale_call(deg0, deg1, x)
    part = _agg_kernel(y, src, dst, zeros_mat)
    out = _dense_call(
        part[0, :N], part[1, :N], y, dinv, x, W, b.reshape(1, D),
        bn_g.reshape(1, D), bn_b.reshape(1, D),
        bn1_g.reshape(1, D), bn1_b.reshape(1, D),
        W1, b1.reshape(1, DFF), W2, b2.reshape(1, D),
        bn2_g.reshape(1, D), bn2_b.reshape(1, D),
    )
    return out
